# Initial kernel scaffold; baseline (speedup 1.0000x reference)
import jax
import jax.numpy as jnp
from jax.experimental import pallas as pl


def kernel(user_item_edges, kg_edge_index, kg_relations, user_emb, entity_emb, relation_emb, W1, b1, W2, b2):
    raise NotImplementedError("write your pallas kernel here")


# speedup vs baseline: 2.4910x; 2.4910x over previous
import functools

import jax
import jax.numpy as jnp
from jax import lax
from jax.experimental import pallas as pl
from jax.experimental.pallas import tpu as pltpu
from jax.experimental.pallas import tpu_sc as plsc

NU = 10000
NEN = 40000
NR = 16
D = 64
NL = 3
N = NU + NEN
E_UI = 200000
E_KG = 200000
E = 2 * E_UI + 2 * E_KG
E_PAD = 819200
EPT = E_PAD // 16
CH = 128
NCHUNK = EPT // CH
NPC = N // 2
ROWS_PT = 1568
ACC_ROWS = 16 * ROWS_PT
TRASH = 25080
N_PAD = 2 * ACC_ROWS
HW = 17
HROWS = ACC_ROWS * HW
HPT = HROWS // 16
PAD_DST = 60000
RCH = 224
RB = 1568

_mesh = plsc.VectorSubcoreMesh(core_axis_name="core", subcore_axis_name="subcore")
_sc_params = pltpu.CompilerParams(use_tc_tiling_on_sc=False)


@functools.partial(
    pl.kernel,
    out_type=jax.ShapeDtypeStruct((2 * HROWS,), jnp.float32),
    mesh=_mesh,
    scratch_types=[
        pltpu.VMEM_SHARED((HROWS,), jnp.float32),
        pltpu.VMEM((CH,), jnp.int32),
        pltpu.VMEM((CH,), jnp.int32),
        pltpu.VMEM((CH,), jnp.int32),
        pltpu.VMEM((CH,), jnp.float32),
        pltpu.VMEM((HPT,), jnp.float32),
    ],
    compiler_params=_sc_params,
)
def _sc_hist(dst_hbm, rel_hbm, zvec_hbm, out_hbm, acc, dstb, relb, idxb, onesb, stage):
    c = lax.axis_index("core")
    s = lax.axis_index("subcore")
    pltpu.sync_copy(zvec_hbm, stage)
    pltpu.sync_copy(stage, acc.at[pl.ds(s * HPT, HPT)])

    @pl.loop(0, CH // 16)
    def _(i):
        onesb[pl.ds(i * 16, 16)] = jnp.full((16,), 1.0, jnp.float32)

    plsc.subcore_barrier()
    lo = c * NPC

    @pl.loop(0, NCHUNK)
    def _(g):
        e0 = s * EPT + g * CH
        pltpu.sync_copy(dst_hbm.at[pl.ds(e0, CH)], dstb)
        pltpu.sync_copy(rel_hbm.at[pl.ds(e0, CH)], relb)

        @pl.loop(0, CH // 16)
        def _(i):
            d = dstb[pl.ds(i * 16, 16)]
            r = relb[pl.ds(i * 16, 16)]
            loc = d - lo
            ok = (loc >= 0) & (loc < NPC)
            li = jnp.where(ok, loc, TRASH)
            idxb[pl.ds(i * 16, 16)] = li * HW + r

        pltpu.sync_copy(onesb, acc.at[idxb], add=True)

    plsc.subcore_barrier()
    pltpu.sync_copy(acc.at[pl.ds(s * HPT, HPT)], stage)
    pltpu.sync_copy(stage, out_hbm.at[pl.ds(c * HROWS + s * HPT, HPT)])


@functools.partial(
    pl.kernel,
    out_type=jax.ShapeDtypeStruct((N_PAD, D), jnp.float32),
    mesh=_mesh,
    scratch_types=[
        pltpu.VMEM_SHARED((ACC_ROWS, D), jnp.float32),
        pltpu.VMEM((CH,), jnp.int32),
        pltpu.VMEM((CH,), jnp.int32),
        pltpu.VMEM((CH,), jnp.int32),
        pltpu.VMEM((CH, D), jnp.float32),
        pltpu.VMEM((RCH, D), jnp.float32),
    ],
    compiler_params=_sc_params,
)
def _sc_agg(src_hbm, dst_hbm, h_hbm, zrows_hbm, out_hbm, acc, srcb, dstb, lib, rowb, stage):
    c = lax.axis_index("core")
    s = lax.axis_index("subcore")
    lo = c * NPC
    pltpu.sync_copy(zrows_hbm, stage)

    @pl.loop(0, ROWS_PT // RCH)
    def _(k):
        pltpu.sync_copy(stage, acc.at[pl.ds(s * ROWS_PT + k * RCH, RCH)])

    plsc.subcore_barrier()

    @pl.loop(0, NCHUNK)
    def _(g):
        e0 = s * EPT + g * CH
        pltpu.sync_copy(src_hbm.at[pl.ds(e0, CH)], srcb)
        pltpu.sync_copy(dst_hbm.at[pl.ds(e0, CH)], dstb)

        @pl.loop(0, CH // 16)
        def _(i):
            d = dstb[pl.ds(i * 16, 16)]
            loc = d - lo
            ok = (loc >= 0) & (loc < NPC)
            lib[pl.ds(i * 16, 16)] = jnp.where(ok, loc, TRASH)

        pltpu.sync_copy(h_hbm.at[srcb], rowb)
        pltpu.sync_copy(rowb, acc.at[lib], add=True)

    plsc.subcore_barrier()

    @pl.loop(0, ROWS_PT // RCH)
    def _(k):
        pltpu.sync_copy(acc.at[pl.ds(s * ROWS_PT + k * RCH, RCH)], stage)
        pltpu.sync_copy(stage,
                        out_hbm.at[pl.ds(c * ACC_ROWS + s * ROWS_PT + k * RCH, RCH)])


def _tc_prep_body(cnt_ref, relp_ref, base_ref, inv_ref):
    cmat = cnt_ref[...]
    base_ref[...] = jnp.dot(cmat, relp_ref[...], preferred_element_type=jnp.float32)
    deg = jnp.sum(cmat, axis=1, keepdims=True)
    inv_ref[...] = 1.0 / jnp.clip(deg, 1.0, None)


def _tc_prep(count_cm, relp):
    return pl.pallas_call(
        _tc_prep_body,
        grid=(N_PAD // RB,),
        in_specs=[
            pl.BlockSpec((RB, HW), lambda i: (i, 0)),
            pl.BlockSpec((HW, D), lambda i: (0, 0)),
        ],
        out_specs=[
            pl.BlockSpec((RB, D), lambda i: (i, 0)),
            pl.BlockSpec((RB, 1), lambda i: (i, 0)),
        ],
        out_shape=[
            jax.ShapeDtypeStruct((N_PAD, D), jnp.float32),
            jax.ShapeDtypeStruct((N_PAD, 1), jnp.float32),
        ],
    )(count_cm, relp)


def _tc_dense_body(h_ref, raw_ref, base_ref, inv_ref, w1_ref, b1_ref, w2_ref,
                   b2_ref, out_ref):
    hp = h_ref[...]
    agg = (raw_ref[...] + base_ref[...]) * inv_ref[...]
    sp = jnp.dot(hp + agg, w1_ref[...], preferred_element_type=jnp.float32) + b1_ref[...]
    bp = jnp.dot(hp * agg, w2_ref[...], preferred_element_type=jnp.float32) + b2_ref[...]
    sp = jnp.where(sp >= 0, sp, 0.2 * sp)
    bp = jnp.where(bp >= 0, bp, 0.2 * bp)
    hl = sp + bp
    nrm = jnp.sqrt(jnp.sum(hl * hl, axis=1, keepdims=True))
    out_ref[...] = hl / jnp.clip(nrm, 1e-12, None)


def _tc_dense(h, raw, base, inv, w1, b1, w2, b2):
    return pl.pallas_call(
        _tc_dense_body,
        grid=(N_PAD // RB,),
        in_specs=[
            pl.BlockSpec((RB, D), lambda i: (i, 0)),
            pl.BlockSpec((RB, D), lambda i: (i, 0)),
            pl.BlockSpec((RB, D), lambda i: (i, 0)),
            pl.BlockSpec((RB, 1), lambda i: (i, 0)),
            pl.BlockSpec((D, D), lambda i: (0, 0)),
            pl.BlockSpec((1, D), lambda i: (0, 0)),
            pl.BlockSpec((D, D), lambda i: (0, 0)),
            pl.BlockSpec((1, D), lambda i: (0, 0)),
        ],
        out_specs=pl.BlockSpec((RB, D), lambda i: (i, 0)),
        out_shape=jax.ShapeDtypeStruct((N_PAD, D), jnp.float32),
    )(h, raw, base, inv, w1, b1, w2, b2)


def kernel(user_item_edges, kg_edge_index, kg_relations, user_emb, entity_emb,
           relation_emb, W1, b1, W2, b2):
    u = user_item_edges[0].astype(jnp.int32)
    it = user_item_edges[1].astype(jnp.int32) + NU
    hd = kg_edge_index[0].astype(jnp.int32) + NU
    tl = kg_edge_index[1].astype(jnp.int32) + NU
    npad = E_PAD - E
    src = jnp.concatenate([u, it, hd, tl, jnp.zeros((npad,), jnp.int32)])
    dst_p = jnp.concatenate([it, u, tl, hd, jnp.full((npad,), PAD_DST, jnp.int32)])
    src_cm = jnp.where(src < NPC, src, src + (ACC_ROWS - NPC))
    kr = kg_relations.astype(jnp.int32)
    rel_full = jnp.concatenate([
        jnp.full((2 * E_UI,), NR, jnp.int32), kr, kr,
        jnp.full((npad,), NR, jnp.int32)])
    zvec = jnp.zeros((HPT,), jnp.float32)
    zrows = jnp.zeros((RCH, D), jnp.float32)

    cnt = _sc_hist(dst_p, rel_full, zvec)
    count_cm = cnt.reshape(N_PAD, HW)
    relp = jnp.concatenate([relation_emb, jnp.zeros((1, D), jnp.float32)], axis=0)
    base, inv = _tc_prep(count_cm, relp)

    h = jnp.concatenate([user_emb, entity_emb], axis=0)
    gap = jnp.zeros((ACC_ROWS - NPC, D), jnp.float32)
    h0 = jnp.concatenate([h[:NPC], gap, h[NPC:], gap], axis=0)
    layers = [h0]
    for l in range(NL):
        raw = _sc_agg(src_cm, dst_p, layers[-1], zrows)
        hn = _tc_dense(layers[-1], raw, base, inv, W1[l], b1[l].reshape(1, D),
                       W2[l], b2[l].reshape(1, D))
        layers.append(hn)
    return jnp.concatenate(
        [jnp.concatenate([x[:NPC], x[ACC_ROWS:ACC_ROWS + NPC]], axis=0)
         for x in layers], axis=-1)


# speedup vs baseline: 2.8953x; 1.1623x over previous
import functools

import jax
import jax.numpy as jnp
from jax import lax
from jax.experimental import pallas as pl
from jax.experimental.pallas import tpu as pltpu
from jax.experimental.pallas import tpu_sc as plsc

NU = 10000
NEN = 40000
NR = 16
D = 64
NL = 3
N = NU + NEN
E_UI = 200000
E_KG = 200000
E = 2 * E_UI + 2 * E_KG
E_PAD = 819200
EPT = E_PAD // 16
CH = 128
NCHUNK = EPT // CH
NPC = N // 2
ROWS_PT = 1568
ACC_ROWS = 16 * ROWS_PT
TRASH = 25080
N_PAD = 2 * ACC_ROWS
HW = 17
HROWS = ACC_ROWS * HW
HPT = HROWS // 16
PAD_DST = 60000
RCH = 56
RB = 1568
BL = 1024
SUB = BL // CH
NBLK = EPT // BL

_mesh = plsc.VectorSubcoreMesh(core_axis_name="core", subcore_axis_name="subcore")
_sc_params = pltpu.CompilerParams(use_tc_tiling_on_sc=False)


@functools.partial(
    pl.kernel,
    out_type=jax.ShapeDtypeStruct((2 * HROWS,), jnp.float32),
    mesh=_mesh,
    scratch_types=[
        pltpu.VMEM_SHARED((HROWS,), jnp.float32),
        pltpu.VMEM((BL,), jnp.int32),
        pltpu.VMEM((BL,), jnp.int32),
        pltpu.VMEM((SUB, CH), jnp.int32),
        pltpu.VMEM((CH,), jnp.float32),
        pltpu.VMEM((HPT,), jnp.float32),
        pltpu.SemaphoreType.DMA,
    ],
    compiler_params=_sc_params,
)
def _sc_hist(dst_hbm, rel_hbm, zvec_hbm, out_hbm, acc, dstb, relb, idxb, onesb,
             stage, ssem):
    c = lax.axis_index("core")
    s = lax.axis_index("subcore")
    pltpu.sync_copy(zvec_hbm, stage)
    pltpu.sync_copy(stage, acc.at[pl.ds(s * HPT, HPT)])

    @pl.loop(0, CH // 16)
    def _(i):
        onesb[pl.ds(i * 16, 16)] = jnp.full((16,), 1.0, jnp.float32)

    plsc.subcore_barrier()
    lo = c * NPC

    @pl.loop(0, NBLK)
    def _(b):
        e0 = s * EPT + b * BL
        pltpu.sync_copy(dst_hbm.at[pl.ds(e0, BL)], dstb)
        pltpu.sync_copy(rel_hbm.at[pl.ds(e0, BL)], relb)

        @pl.loop(0, SUB)
        def _(k):
            @pl.loop(0, CH // 16)
            def _(i):
                d = dstb[pl.ds(k * CH + i * 16, 16)]
                r = relb[pl.ds(k * CH + i * 16, 16)]
                loc = d - lo
                ok = (loc >= 0) & (loc < NPC)
                li = jnp.where(ok, loc, TRASH)
                idxb.at[pl.ds(k, 1), pl.ds(i * 16, 16)][...] = \
                    (li * HW + r).reshape(1, 16)

        for k in range(SUB):
            pltpu.async_copy(onesb, acc.at[idxb.at[k]], ssem, add=True)
        for k in range(SUB):
            pltpu.make_async_copy(onesb, acc.at[idxb.at[k]], ssem).wait()

    plsc.subcore_barrier()
    pltpu.sync_copy(acc.at[pl.ds(s * HPT, HPT)], stage)
    pltpu.sync_copy(stage, out_hbm.at[pl.ds(c * HROWS + s * HPT, HPT)])


@functools.partial(
    pl.kernel,
    out_type=jax.ShapeDtypeStruct((N_PAD, D), jnp.float32),
    mesh=_mesh,
    scratch_types=[
        pltpu.VMEM_SHARED((ACC_ROWS, D), jnp.float32),
        pltpu.VMEM((BL,), jnp.int32),
        pltpu.VMEM((BL,), jnp.int32),
        pltpu.VMEM((SUB, CH), jnp.int32),
        pltpu.VMEM((2, CH, D), jnp.float32),
        pltpu.VMEM((RCH, D), jnp.float32),
        pltpu.SemaphoreType.DMA,
        pltpu.SemaphoreType.DMA,
    ],
    compiler_params=_sc_params,
)
def _sc_agg(src_hbm, dst_hbm, h_hbm, zrows_hbm, out_hbm, acc, srcb, dstb, lib,
            rowb, stage, gsem0, gsem1):
    c = lax.axis_index("core")
    s = lax.axis_index("subcore")
    lo = c * NPC
    pltpu.sync_copy(zrows_hbm, stage)

    @pl.loop(0, ROWS_PT // RCH)
    def _(k):
        pltpu.sync_copy(stage, acc.at[pl.ds(s * ROWS_PT + k * RCH, RCH)])

    plsc.subcore_barrier()

    @pl.loop(0, NBLK)
    def _(b):
        e0 = s * EPT + b * BL
        pltpu.sync_copy(src_hbm.at[pl.ds(e0, BL)], srcb)
        pltpu.sync_copy(dst_hbm.at[pl.ds(e0, BL)], dstb)

        @pl.loop(0, SUB)
        def _(k):
            @pl.loop(0, CH // 16)
            def _(i):
                d = dstb[pl.ds(k * CH + i * 16, 16)]
                loc = d - lo
                ok = (loc >= 0) & (loc < NPC)
                li = jnp.where(ok, loc, TRASH)
                lib.at[pl.ds(k, 1), pl.ds(i * 16, 16)][...] = li.reshape(1, 16)

        def _issue_gather(k, buf):
            sem = gsem0 if buf == 0 else gsem1
            pltpu.async_copy(h_hbm.at[srcb.at[pl.ds(k * CH, CH)]],
                             rowb.at[buf], sem)

        _issue_gather(0, 0)
        for k in range(SUB):
            bsel = k % 2
            sem = gsem0 if bsel == 0 else gsem1
            pltpu.make_async_copy(h_hbm.at[srcb.at[pl.ds(k * CH, CH)]],
                                  rowb.at[bsel], sem).wait()
            if k + 1 < SUB:
                _issue_gather(k + 1, 1 - bsel)
            pltpu.sync_copy(rowb.at[bsel], acc.at[lib.at[k]], add=True)

    plsc.subcore_barrier()

    @pl.loop(0, ROWS_PT // RCH)
    def _(k):
        pltpu.sync_copy(acc.at[pl.ds(s * ROWS_PT + k * RCH, RCH)], stage)
        pltpu.sync_copy(stage,
                        out_hbm.at[pl.ds(c * ACC_ROWS + s * ROWS_PT + k * RCH, RCH)])


def _tc_prep_body(cnt_ref, relp_ref, base_ref, inv_ref):
    cmat = cnt_ref[...]
    base_ref[...] = jnp.dot(cmat, relp_ref[...], preferred_element_type=jnp.float32)
    deg = jnp.sum(cmat, axis=1, keepdims=True)
    inv_ref[...] = 1.0 / jnp.clip(deg, 1.0, None)


def _tc_prep(count_cm, relp):
    return pl.pallas_call(
        _tc_prep_body,
        grid=(N_PAD // RB,),
        in_specs=[
            pl.BlockSpec((RB, HW), lambda i: (i, 0)),
            pl.BlockSpec((HW, D), lambda i: (0, 0)),
        ],
        out_specs=[
            pl.BlockSpec((RB, D), lambda i: (i, 0)),
            pl.BlockSpec((RB, 1), lambda i: (i, 0)),
        ],
        out_shape=[
            jax.ShapeDtypeStruct((N_PAD, D), jnp.float32),
            jax.ShapeDtypeStruct((N_PAD, 1), jnp.float32),
        ],
    )(count_cm, relp)


def _tc_dense_body(h_ref, raw_ref, base_ref, inv_ref, w1_ref, b1_ref, w2_ref,
                   b2_ref, out_ref):
    hp = h_ref[...]
    agg = (raw_ref[...] + base_ref[...]) * inv_ref[...]
    sp = jnp.dot(hp + agg, w1_ref[...], preferred_element_type=jnp.float32) + b1_ref[...]
    bp = jnp.dot(hp * agg, w2_ref[...], preferred_element_type=jnp.float32) + b2_ref[...]
    sp = jnp.where(sp >= 0, sp, 0.2 * sp)
    bp = jnp.where(bp >= 0, bp, 0.2 * bp)
    hl = sp + bp
    nrm = jnp.sqrt(jnp.sum(hl * hl, axis=1, keepdims=True))
    out_ref[...] = hl / jnp.clip(nrm, 1e-12, None)


def _tc_dense(h, raw, base, inv, w1, b1, w2, b2):
    return pl.pallas_call(
        _tc_dense_body,
        grid=(N_PAD // RB,),
        in_specs=[
            pl.BlockSpec((RB, D), lambda i: (i, 0)),
            pl.BlockSpec((RB, D), lambda i: (i, 0)),
            pl.BlockSpec((RB, D), lambda i: (i, 0)),
            pl.BlockSpec((RB, 1), lambda i: (i, 0)),
            pl.BlockSpec((D, D), lambda i: (0, 0)),
            pl.BlockSpec((1, D), lambda i: (0, 0)),
            pl.BlockSpec((D, D), lambda i: (0, 0)),
            pl.BlockSpec((1, D), lambda i: (0, 0)),
        ],
        out_specs=pl.BlockSpec((RB, D), lambda i: (i, 0)),
        out_shape=jax.ShapeDtypeStruct((N_PAD, D), jnp.float32),
    )(h, raw, base, inv, w1, b1, w2, b2)


def kernel(user_item_edges, kg_edge_index, kg_relations, user_emb, entity_emb,
           relation_emb, W1, b1, W2, b2):
    u = user_item_edges[0].astype(jnp.int32)
    it = user_item_edges[1].astype(jnp.int32) + NU
    hd = kg_edge_index[0].astype(jnp.int32) + NU
    tl = kg_edge_index[1].astype(jnp.int32) + NU
    npad = E_PAD - E
    src = jnp.concatenate([u, it, hd, tl, jnp.zeros((npad,), jnp.int32)])
    dst_p = jnp.concatenate([it, u, tl, hd, jnp.full((npad,), PAD_DST, jnp.int32)])
    src_cm = jnp.where(src < NPC, src, src + (ACC_ROWS - NPC))
    kr = kg_relations.astype(jnp.int32)
    rel_full = jnp.concatenate([
        jnp.full((2 * E_UI,), NR, jnp.int32), kr, kr,
        jnp.full((npad,), NR, jnp.int32)])
    zvec = jnp.zeros((HPT,), jnp.float32)
    zrows = jnp.zeros((RCH, D), jnp.float32)

    cnt = _sc_hist(dst_p, rel_full, zvec)
    count_cm = cnt.reshape(N_PAD, HW)
    relp = jnp.concatenate([relation_emb, jnp.zeros((1, D), jnp.float32)], axis=0)
    base, inv = _tc_prep(count_cm, relp)

    h = jnp.concatenate([user_emb, entity_emb], axis=0)
    gap = jnp.zeros((ACC_ROWS - NPC, D), jnp.float32)
    h0 = jnp.concatenate([h[:NPC], gap, h[NPC:], gap], axis=0)
    layers = [h0]
    for l in range(NL):
        raw = _sc_agg(src_cm, dst_p, layers[-1], zrows)
        hn = _tc_dense(layers[-1], raw, base, inv, W1[l], b1[l].reshape(1, D),
                       W2[l], b2[l].reshape(1, D))
        layers.append(hn)
    return jnp.concatenate(
        [jnp.concatenate([x[:NPC], x[ACC_ROWS:ACC_ROWS + NPC]], axis=0)
         for x in layers], axis=-1)


# speedup vs baseline: 2.9205x; 1.0087x over previous
import functools

import jax
import jax.numpy as jnp
from jax import lax
from jax.experimental import pallas as pl
from jax.experimental.pallas import tpu as pltpu
from jax.experimental.pallas import tpu_sc as plsc

NU = 10000
NEN = 40000
NR = 16
D = 64
NL = 3
N = NU + NEN
E_UI = 200000
E_KG = 200000
E = 2 * E_UI + 2 * E_KG
E_PAD = 819200
EPT = E_PAD // 16
CH = 256
NPC = N // 2
ROWS_PT = 1568
ACC_ROWS = 16 * ROWS_PT
TRASH = 25080
N_PAD = 2 * ACC_ROWS
HW = 17
HROWS = ACC_ROWS * HW
HPT = HROWS // 16
PAD_DST = 60000
WCH = 112
RB = 1568
BL = 2048
SUB = BL // CH
NBLK = EPT // BL

_mesh = plsc.VectorSubcoreMesh(core_axis_name="core", subcore_axis_name="subcore")
_sc_params = pltpu.CompilerParams(use_tc_tiling_on_sc=False)


@functools.partial(
    pl.kernel,
    out_type=jax.ShapeDtypeStruct((2 * HROWS,), jnp.float32),
    mesh=_mesh,
    scratch_types=[
        pltpu.VMEM_SHARED((HROWS,), jnp.float32),
        pltpu.VMEM((BL,), jnp.int32),
        pltpu.VMEM((BL,), jnp.int32),
        pltpu.VMEM((SUB, CH), jnp.int32),
        pltpu.VMEM((CH,), jnp.float32),
        pltpu.VMEM((HPT,), jnp.float32),
        pltpu.SemaphoreType.DMA,
    ],
    compiler_params=_sc_params,
)
def _sc_hist(dst_hbm, rel_hbm, zvec_hbm, out_hbm, acc, dstb, relb, idxb, onesb,
             stage, ssem):
    c = lax.axis_index("core")
    s = lax.axis_index("subcore")
    pltpu.sync_copy(zvec_hbm, stage)
    pltpu.sync_copy(stage, acc.at[pl.ds(s * HPT, HPT)])

    @pl.loop(0, CH // 16)
    def _(i):
        onesb[pl.ds(i * 16, 16)] = jnp.full((16,), 1.0, jnp.float32)

    plsc.subcore_barrier()
    lo = c * NPC

    @pl.loop(0, NBLK)
    def _(b):
        e0 = s * EPT + b * BL
        pltpu.sync_copy(dst_hbm.at[pl.ds(e0, BL)], dstb)
        pltpu.sync_copy(rel_hbm.at[pl.ds(e0, BL)], relb)

        @pl.loop(0, SUB)
        def _(k):
            @pl.loop(0, CH // 16)
            def _(i):
                d = dstb[pl.ds(k * CH + i * 16, 16)]
                r = relb[pl.ds(k * CH + i * 16, 16)]
                loc = d - lo
                ok = (loc >= 0) & (loc < NPC)
                li = jnp.where(ok, loc, TRASH)
                idxb.at[pl.ds(k, 1), pl.ds(i * 16, 16)][...] = \
                    (li * HW + r).reshape(1, 16)

        for k in range(SUB):
            pltpu.async_copy(onesb, acc.at[idxb.at[k]], ssem, add=True)
        for k in range(SUB):
            pltpu.make_async_copy(onesb, acc.at[idxb.at[k]], ssem).wait()

    plsc.subcore_barrier()
    pltpu.sync_copy(acc.at[pl.ds(s * HPT, HPT)], stage)
    pltpu.sync_copy(stage, out_hbm.at[pl.ds(c * HROWS + s * HPT, HPT)])


@functools.partial(
    pl.kernel,
    out_type=jax.ShapeDtypeStruct((N_PAD, D), jnp.float32),
    mesh=_mesh,
    scratch_types=[
        pltpu.VMEM_SHARED((ACC_ROWS, D), jnp.float32),
        pltpu.VMEM((BL,), jnp.int32),
        pltpu.VMEM((BL,), jnp.int32),
        pltpu.VMEM((SUB, CH), jnp.int32),
        pltpu.VMEM((CH, D), jnp.float32),
        pltpu.SemaphoreType.DMA,
    ],
    compiler_params=_sc_params,
)
def _sc_agg(src_hbm, dst_hbm, h_hbm, zrows_hbm, out_hbm, acc, srcb, dstb, lib,
            rowb, isem):
    c = lax.axis_index("core")
    s = lax.axis_index("subcore")
    lo = c * NPC
    pltpu.sync_copy(zrows_hbm, rowb.at[pl.ds(0, WCH)])

    @pl.loop(0, ROWS_PT // WCH)
    def _(k):
        pltpu.async_copy(rowb.at[pl.ds(0, WCH)],
                         acc.at[pl.ds(s * ROWS_PT + k * WCH, WCH)], isem)

    @pl.loop(0, ROWS_PT // WCH)
    def _(k):
        pltpu.make_async_copy(rowb.at[pl.ds(0, WCH)],
                              acc.at[pl.ds(s * ROWS_PT + k * WCH, WCH)],
                              isem).wait()

    plsc.subcore_barrier()

    @pl.loop(0, NBLK)
    def _(b):
        e0 = s * EPT + b * BL
        pltpu.sync_copy(src_hbm.at[pl.ds(e0, BL)], srcb)
        pltpu.sync_copy(dst_hbm.at[pl.ds(e0, BL)], dstb)

        @pl.loop(0, SUB)
        def _(k):
            @pl.loop(0, CH // 16)
            def _(i):
                d = dstb[pl.ds(k * CH + i * 16, 16)]
                loc = d - lo
                ok = (loc >= 0) & (loc < NPC)
                li = jnp.where(ok, loc, TRASH)
                lib.at[pl.ds(k, 1), pl.ds(i * 16, 16)][...] = li.reshape(1, 16)

        for k in range(SUB):
            pltpu.sync_copy(h_hbm.at[srcb.at[pl.ds(k * CH, CH)]], rowb)
            pltpu.sync_copy(rowb, acc.at[lib.at[k]], add=True)

    plsc.subcore_barrier()

    @pl.loop(0, ROWS_PT // WCH)
    def _(k):
        pltpu.sync_copy(acc.at[pl.ds(s * ROWS_PT + k * WCH, WCH)],
                        rowb.at[pl.ds(0, WCH)])
        pltpu.sync_copy(rowb.at[pl.ds(0, WCH)],
                        out_hbm.at[pl.ds(c * ACC_ROWS + s * ROWS_PT + k * WCH, WCH)])


def _tc_prep_body(cnt_ref, relp_ref, base_ref, inv_ref):
    cmat = cnt_ref[...]
    base_ref[...] = jnp.dot(cmat, relp_ref[...], preferred_element_type=jnp.float32)
    deg = jnp.sum(cmat, axis=1, keepdims=True)
    inv_ref[...] = 1.0 / jnp.clip(deg, 1.0, None)


def _tc_prep(count_cm, relp):
    return pl.pallas_call(
        _tc_prep_body,
        grid=(N_PAD // RB,),
        in_specs=[
            pl.BlockSpec((RB, HW), lambda i: (i, 0)),
            pl.BlockSpec((HW, D), lambda i: (0, 0)),
        ],
        out_specs=[
            pl.BlockSpec((RB, D), lambda i: (i, 0)),
            pl.BlockSpec((RB, 1), lambda i: (i, 0)),
        ],
        out_shape=[
            jax.ShapeDtypeStruct((N_PAD, D), jnp.float32),
            jax.ShapeDtypeStruct((N_PAD, 1), jnp.float32),
        ],
    )(count_cm, relp)


def _tc_dense_body(h_ref, raw_ref, base_ref, inv_ref, w1_ref, b1_ref, w2_ref,
                   b2_ref, out_ref):
    hp = h_ref[...]
    agg = (raw_ref[...] + base_ref[...]) * inv_ref[...]
    sp = jnp.dot(hp + agg, w1_ref[...], preferred_element_type=jnp.float32) + b1_ref[...]
    bp = jnp.dot(hp * agg, w2_ref[...], preferred_element_type=jnp.float32) + b2_ref[...]
    sp = jnp.where(sp >= 0, sp, 0.2 * sp)
    bp = jnp.where(bp >= 0, bp, 0.2 * bp)
    hl = sp + bp
    nrm = jnp.sqrt(jnp.sum(hl * hl, axis=1, keepdims=True))
    out_ref[...] = hl / jnp.clip(nrm, 1e-12, None)


def _tc_dense(h, raw, base, inv, w1, b1, w2, b2):
    return pl.pallas_call(
        _tc_dense_body,
        grid=(N_PAD // RB,),
        in_specs=[
            pl.BlockSpec((RB, D), lambda i: (i, 0)),
            pl.BlockSpec((RB, D), lambda i: (i, 0)),
            pl.BlockSpec((RB, D), lambda i: (i, 0)),
            pl.BlockSpec((RB, 1), lambda i: (i, 0)),
            pl.BlockSpec((D, D), lambda i: (0, 0)),
            pl.BlockSpec((1, D), lambda i: (0, 0)),
            pl.BlockSpec((D, D), lambda i: (0, 0)),
            pl.BlockSpec((1, D), lambda i: (0, 0)),
        ],
        out_specs=pl.BlockSpec((RB, D), lambda i: (i, 0)),
        out_shape=jax.ShapeDtypeStruct((N_PAD, D), jnp.float32),
    )(h, raw, base, inv, w1, b1, w2, b2)


def kernel(user_item_edges, kg_edge_index, kg_relations, user_emb, entity_emb,
           relation_emb, W1, b1, W2, b2):
    u = user_item_edges[0].astype(jnp.int32)
    it = user_item_edges[1].astype(jnp.int32) + NU
    hd = kg_edge_index[0].astype(jnp.int32) + NU
    tl = kg_edge_index[1].astype(jnp.int32) + NU
    npad = E_PAD - E
    src = jnp.concatenate([u, it, hd, tl, jnp.zeros((npad,), jnp.int32)])
    dst_p = jnp.concatenate([it, u, tl, hd, jnp.full((npad,), PAD_DST, jnp.int32)])
    src_cm = jnp.where(src < NPC, src, src + (ACC_ROWS - NPC))
    kr = kg_relations.astype(jnp.int32)
    rel_full = jnp.concatenate([
        jnp.full((2 * E_UI,), NR, jnp.int32), kr, kr,
        jnp.full((npad,), NR, jnp.int32)])
    zvec = jnp.zeros((HPT,), jnp.float32)
    zrows = jnp.zeros((WCH, D), jnp.float32)

    cnt = _sc_hist(dst_p, rel_full, zvec)
    count_cm = cnt.reshape(N_PAD, HW)
    relp = jnp.concatenate([relation_emb, jnp.zeros((1, D), jnp.float32)], axis=0)
    base, inv = _tc_prep(count_cm, relp)

    h = jnp.concatenate([user_emb, entity_emb], axis=0)
    gap = jnp.zeros((ACC_ROWS - NPC, D), jnp.float32)
    h0 = jnp.concatenate([h[:NPC], gap, h[NPC:], gap], axis=0)
    layers = [h0]
    for l in range(NL):
        raw = _sc_agg(src_cm, dst_p, layers[-1], zrows)
        hn = _tc_dense(layers[-1], raw, base, inv, W1[l], b1[l].reshape(1, D),
                       W2[l], b2[l].reshape(1, D))
        layers.append(hn)
    return jnp.concatenate(
        [jnp.concatenate([x[:NPC], x[ACC_ROWS:ACC_ROWS + NPC]], axis=0)
         for x in layers], axis=-1)
